# baseline (speedup 1.0000x reference)
import functools
import math

import jax
import jax.numpy as jnp
from jax import lax
from jax.experimental import pallas as pl
from jax.experimental.pallas import tpu as pltpu
from jax.experimental.pallas import tpu_sc as plsc

_D = 128
_SCALE = math.sqrt(128.0)
_B = 16384 * 50
_NW = 32
_BPW = _B // _NW
_C = 128
_NCHUNK = _BPW // _C
_LANES = 16


def _scale_chunk(rows_ref, b):
    def row_body(r, carry):
        for j in range(_D // _LANES):
            sl = (b, r, pl.ds(j * _LANES, _LANES))
            rows_ref[sl] = rows_ref[sl] * _SCALE
        return carry
    lax.fori_loop(0, _C, row_body, 0)


def _body(x_hbm, lut_hbm, out_hbm, idx_v, rows_v, gsem, osem):
    wid = lax.axis_index("s") * 2 + lax.axis_index("c")
    base = wid * _BPW

    def stage_and_gather(c, buf):
        pltpu.sync_copy(x_hbm.at[pl.ds(base + c * _C, _C)], idx_v.at[buf])
        pltpu.async_copy(lut_hbm.at[idx_v.at[buf]], rows_v.at[buf], gsem)

    def gather_wait(buf):
        pltpu.make_async_copy(
            lut_hbm.at[idx_v.at[buf]], rows_v.at[buf], gsem).wait()

    def out_start(c, buf):
        pltpu.async_copy(
            rows_v.at[buf], out_hbm.at[pl.ds(base + c * _C, _C)], osem)

    def out_wait(c, buf):
        pltpu.make_async_copy(
            rows_v.at[buf], out_hbm.at[pl.ds(base + c * _C, _C)], osem).wait()

    stage_and_gather(0, 0)

    def loop_body(c0, carry):
        for buf in range(2):
            c = c0 + buf
            nxt = c + 1
            gather_wait(buf)
            nbuf = buf ^ 1

            @pl.when(c >= 1)
            def _wait_prev():
                out_wait(c - 1, nbuf)

            @pl.when(nxt < _NCHUNK)
            def _issue_next():
                stage_and_gather(nxt, nbuf)

            _scale_chunk(rows_v, buf)
            out_start(c, buf)
        return carry

    lax.fori_loop(0, _NCHUNK // 2, lambda i, a: loop_body(i * 2, a), 0)
    out_wait(_NCHUNK - 1, 1)


@jax.jit
def _lookup(x_flat, lut):
    mesh = plsc.VectorSubcoreMesh(core_axis_name="c", subcore_axis_name="s")
    f = functools.partial(
        pl.kernel,
        mesh=mesh,
        out_type=jax.ShapeDtypeStruct((_B, _D), jnp.float32),
        scratch_types=[
            pltpu.VMEM((2, _C), jnp.int32),
            pltpu.VMEM((2, _C, _D), jnp.float32),
            pltpu.SemaphoreType.DMA,
            pltpu.SemaphoreType.DMA,
        ],
    )(_body)
    return f(x_flat, lut)


def kernel(x, lut):
    x_flat = x.reshape(-1).astype(jnp.int32)
    out = _lookup(x_flat, lut)
    return out.reshape(x.shape[0], x.shape[1], _D)


# speedup vs baseline: 1.1256x; 1.1256x over previous
import functools
import math

import jax
import jax.numpy as jnp
from jax import lax
from jax.experimental import pallas as pl
from jax.experimental.pallas import tpu as pltpu
from jax.experimental.pallas import tpu_sc as plsc

_D = 128
_SCALE = math.sqrt(128.0)
_B = 16384 * 50
_NW = 32
_BPW = _B // _NW
_C = 128
_NCHUNK = _BPW // _C
_NBUF = 4
_LANES = 16
_RU = 4


def _scale_chunk(rows_ref, buf):
    def row_body(r0, carry):
        for u in range(_RU):
            for j in range(_D // _LANES):
                sl = (buf, r0 * _RU + u, pl.ds(j * _LANES, _LANES))
                rows_ref[sl] = rows_ref[sl] * _SCALE
        return carry
    lax.fori_loop(0, _C // _RU, row_body, 0)


def _body(x_hbm, lut_hbm, out_hbm, idx_all, rows_v, gsem, osem):
    wid = lax.axis_index("s") * 2 + lax.axis_index("c")
    base = wid * _BPW

    pltpu.sync_copy(x_hbm.at[pl.ds(wid * _NCHUNK, _NCHUNK)], idx_all)

    def gather_start(c, buf):
        pltpu.async_copy(lut_hbm.at[idx_all.at[c]], rows_v.at[buf], gsem)

    def gather_wait(c, buf):
        pltpu.make_async_copy(
            lut_hbm.at[idx_all.at[c]], rows_v.at[buf], gsem).wait()

    def out_start(c, buf):
        pltpu.async_copy(
            rows_v.at[buf], out_hbm.at[pl.ds(base + c * _C, _C)], osem)

    def out_wait(c, buf):
        pltpu.make_async_copy(
            rows_v.at[buf], out_hbm.at[pl.ds(base + c * _C, _C)], osem).wait()

    for c in range(_NBUF - 1):
        gather_start(c, c)

    def loop_body(c0, carry):
        for buf in range(_NBUF):
            c = c0 + buf
            gather_wait(c, buf)
            _scale_chunk(rows_v, buf)
            out_start(c, buf)

            nxt = c + _NBUF - 1
            nbuf = (buf + _NBUF - 1) % _NBUF

            @pl.when(c >= 1)
            def _drain_prev_out():
                out_wait(c - 1, nbuf)

            @pl.when(nxt < _NCHUNK)
            def _issue_next():
                gather_start(nxt, nbuf)
        return carry

    lax.fori_loop(0, _NCHUNK // _NBUF,
                  lambda i, a: loop_body(i * _NBUF, a), 0)
    out_wait(_NCHUNK - 1, _NBUF - 1)


@jax.jit
def _lookup(x_2d, lut):
    mesh = plsc.VectorSubcoreMesh(core_axis_name="c", subcore_axis_name="s")
    f = functools.partial(
        pl.kernel,
        mesh=mesh,
        out_type=jax.ShapeDtypeStruct((_B, _D), jnp.float32),
        scratch_types=[
            pltpu.VMEM((_NCHUNK, _C), jnp.int32),
            pltpu.VMEM((_NBUF, _C, _D), jnp.float32),
            pltpu.SemaphoreType.DMA,
            pltpu.SemaphoreType.DMA,
        ],
    )(_body)
    return f(x_2d, lut)


def kernel(x, lut):
    x_2d = x.reshape(_B // _C, _C).astype(jnp.int32)
    out = _lookup(x_2d, lut)
    return out.reshape(x.shape[0], x.shape[1], _D)
